# Initial kernel scaffold; baseline (speedup 1.0000x reference)
import jax
import jax.numpy as jnp
from jax.experimental import pallas as pl


def kernel(depth, context, input_feature_map, bin_num, min_depth, max_depth, masks, instances, boxes, labels, scale_W1, scale_b1, scale_W2, scale_b2, scale_fc_w, scale_fc_b, canon_W1, canon_b1, canon_W2, canon_b2):
    raise NotImplementedError("write your pallas kernel here")


# speedup vs baseline: 1.3008x; 1.3008x over previous
import jax
import jax.numpy as jnp
from jax import lax
from jax.experimental import pallas as pl
from jax.experimental.pallas import tpu as pltpu

_B, _I, _D, _C = 2, 16, 128, 13
_HH = 32
_HO = 128
_P = _HH * _HH


def _shifted(Xm, xpos, dy, dx):
    o = dy * _HH + dx
    n = Xm.shape[1]
    if o > 0:
        Xs = jnp.concatenate([Xm[o:, :], jnp.zeros((o, n), jnp.float32)], axis=0)
    elif o < 0:
        Xs = jnp.concatenate([jnp.zeros((-o, n), jnp.float32), Xm[:o, :]], axis=0)
    else:
        Xs = Xm
    if dx == -1:
        Xs = jnp.where(xpos > 0, Xs, 0.0)
    elif dx == 1:
        Xs = jnp.where(xpos < _HH - 1, Xs, 0.0)
    return Xs


def _conv_mm(Xm, xpos, w_ref, b, n_out):
    acc = jnp.zeros((_P, n_out), jnp.float32)
    t = 0
    for dy in (-1, 0, 1):
        for dx in (-1, 0, 1):
            Xs = _shifted(Xm, xpos, dy, dx)
            acc = acc + jnp.dot(Xs, w_ref[t * _D:(t + 1) * _D, :],
                                preferred_element_type=jnp.float32)
            t += 1
    return acc + b[None, :]


def _stage1_body(x_ref, w1_ref, b1_ref, w2_ref, b2_ref, wca2_ref, bca2_ref,
                 fcw_ref, fcb_ref, a_ref, at_ref, up_ref, ss_ref):
    X = x_ref[0]
    xpos = lax.broadcasted_iota(jnp.int32, (_P, 1), 0) % _HH
    h = jnp.maximum(_conv_mm(X, xpos, w1_ref[0], b1_ref[0, 0], 2 * _D), 0.0)
    sc1 = h[:, :_D]
    ca1 = h[:, _D:]
    sc2 = jnp.maximum(_conv_mm(sc1, xpos, w2_ref[0], b2_ref[0, 0], _D), 0.0)
    pooled = jnp.mean(sc2, axis=0)
    ssw = jnp.dot(pooled, fcw_ref[0], preferred_element_type=jnp.float32) \
        + fcb_ref[0, 0]
    ss_ref[0, 0] = jnp.broadcast_to(ssw[None, :], (8, _D))
    w = wca2_ref[0, 0]
    c2 = jnp.zeros((_P,), jnp.float32)
    t = 0
    for dy in (-1, 0, 1):
        for dx in (-1, 0, 1):
            Xs = _shifted(ca1, xpos, dy, dx)
            c2 = c2 + jnp.sum(Xs * w[t * _D:(t + 1) * _D][None, :], axis=1)
            t += 1
    c2 = c2 + bca2_ref[0, 0, 0]
    c2m = c2.reshape(_HH, _HH)
    up = jnp.dot(jnp.dot(a_ref[...], c2m, preferred_element_type=jnp.float32),
                 at_ref[...], preferred_element_type=jnp.float32)
    up_ref[0, 0] = up


def _stage2_body(pair_ref, lab_ref, up_ref, ss_ref, canon_ref, depth_ref):
    j = pl.program_id(0)
    valid = lab_ref[j] > 0
    up = up_ref[0]
    s = ss_ref[0, 0, 0]
    t = ss_ref[0, 0, 1]
    canon = jnp.where(valid, up, 0.0)
    sv = jnp.where(valid, s, 0.0)
    tv = jnp.where(valid, t, 0.0)
    canon_ref[0] = canon
    depth_ref[0] = jnp.maximum(canon * sv + tv, 0.001)


def _to_mm(W):
    C, O, Iin = W.shape[0], W.shape[1], W.shape[2]
    return W.transpose(0, 3, 4, 2, 1).reshape(C, 9 * Iin, O)


def kernel(depth, context, input_feature_map, bin_num, min_depth, max_depth,
           masks, instances, boxes, labels,
           scale_W1, scale_b1, scale_W2, scale_b2, scale_fc_w, scale_fc_b,
           canon_W1, canon_b1, canon_W2, canon_b2):
    f32 = jnp.float32
    b, _, h, w = depth.shape
    X = input_feature_map.transpose(0, 2, 3, 1).reshape(_B, _P, _D)

    w1cat = jnp.concatenate([_to_mm(scale_W1), _to_mm(canon_W1)], axis=2)
    b1cat = jnp.broadcast_to(
        jnp.concatenate([scale_b1, canon_b1], axis=1)[:, None, :], (_C, 8, 2 * _D))
    w2 = _to_mm(scale_W2)
    b2 = jnp.broadcast_to(scale_b2[:, None, :], (_C, 8, _D))
    wca2 = jnp.broadcast_to(_to_mm(canon_W2)[:, None, :, 0], (_C, 8, 9 * _D))
    bca2 = jnp.broadcast_to(canon_b2[:, :, None], (_C, 8, _D))
    fcw = jnp.pad(scale_fc_w, ((0, 0), (0, 0), (0, _D - 2)))
    fcb = jnp.broadcast_to(
        jnp.pad(scale_fc_b, ((0, 0), (0, _D - 2)))[:, None, :], (_C, 8, _D))
    A = jax.image.resize(jnp.eye(_HH, dtype=f32), (_HO, _HH), 'bilinear')
    At = jnp.asarray(A.T)

    grid1 = (_C, _B)
    up_all, ss_all = pl.pallas_call(
        _stage1_body,
        grid=grid1,
        in_specs=[
            pl.BlockSpec((1, _P, _D), lambda c, s: (s, 0, 0)),
            pl.BlockSpec((1, 9 * _D, 2 * _D), lambda c, s: (c, 0, 0)),
            pl.BlockSpec((1, 8, 2 * _D), lambda c, s: (c, 0, 0)),
            pl.BlockSpec((1, 9 * _D, _D), lambda c, s: (c, 0, 0)),
            pl.BlockSpec((1, 8, _D), lambda c, s: (c, 0, 0)),
            pl.BlockSpec((1, 8, 9 * _D), lambda c, s: (c, 0, 0)),
            pl.BlockSpec((1, 8, _D), lambda c, s: (c, 0, 0)),
            pl.BlockSpec((1, _D, _D), lambda c, s: (c, 0, 0)),
            pl.BlockSpec((1, 8, _D), lambda c, s: (c, 0, 0)),
            pl.BlockSpec((_HO, _HH), lambda c, s: (0, 0)),
            pl.BlockSpec((_HH, _HO), lambda c, s: (0, 0)),
        ],
        out_specs=[
            pl.BlockSpec((1, 1, _HO, _HO), lambda c, s: (c, s, 0, 0)),
            pl.BlockSpec((1, 1, 8, _D), lambda c, s: (c, s, 0, 0)),
        ],
        out_shape=[
            jax.ShapeDtypeStruct((_C, _B, _HO, _HO), f32),
            jax.ShapeDtypeStruct((_C, _B, 8, _D), f32),
        ],
    )(X, w1cat, b1cat, w2, b2, wca2, bca2, fcw, fcb, A, At)

    labf = labels.reshape(_B * _I).astype(jnp.int32)
    b_of = (jnp.arange(_B * _I, dtype=jnp.int32) // _I)
    pair = jnp.clip(labf - 1, 0, _C - 1) * _B + b_of

    up2 = up_all.reshape(_C * _B, _HO, _HO)
    ss2 = ss_all.reshape(_C * _B, 8, _D)

    canon, dep = pl.pallas_call(
        _stage2_body,
        grid_spec=pltpu.PrefetchScalarGridSpec(
            num_scalar_prefetch=2,
            grid=(_B * _I,),
            in_specs=[
                pl.BlockSpec((1, _HO, _HO), lambda j, p, l: (p[j], 0, 0)),
                pl.BlockSpec((1, 8, _D), lambda j, p, l: (p[j], 0, 0)),
            ],
            out_specs=[
                pl.BlockSpec((1, _HO, _HO), lambda j, p, l: (j, 0, 0)),
                pl.BlockSpec((1, _HO, _HO), lambda j, p, l: (j, 0, 0)),
            ],
        ),
        out_shape=[
            jax.ShapeDtypeStruct((_B * _I, _HO, _HO), f32),
            jax.ShapeDtypeStruct((_B * _I, _HO, _HO), f32),
        ],
    )(pair, labf, up2, ss2)

    lab2 = labels.astype(jnp.int32)
    li = jnp.clip(lab2 - 1, 0, _C - 1)
    s_bt = ss_all[:, :, 0, 0].transpose(1, 0)
    t_bt = ss_all[:, :, 0, 1].transpose(1, 0)
    s_out = jnp.where(lab2 > 0, jnp.take_along_axis(s_bt, li, axis=1), 0.0)
    t_out = jnp.where(lab2 > 0, jnp.take_along_axis(t_bt, li, axis=1), 0.0)

    return (dep.reshape(_B, _I, _HO, _HO),
            canon.reshape(_B, _I, _HO, _HO),
            s_out, t_out)
